# Initial kernel scaffold; baseline (speedup 1.0000x reference)
import jax
import jax.numpy as jnp
from jax.experimental import pallas as pl


def kernel(node_emb, edge_index, edge_label):
    raise NotImplementedError("write your pallas kernel here")


# speedup vs baseline: 1.8155x; 1.8155x over previous
import functools

import jax
import jax.numpy as jnp
from jax import lax
from jax.experimental import pallas as pl
from jax.experimental.pallas import tpu as pltpu
from jax.experimental.pallas import tpu_sc as plsc

N = 100000
D = 128
B = 500000

NC = 2
NS = 16
NW = NC * NS
L = 16

C = 128
K = -(-B // (NW * C))
BP = NW * C * K
EPW = C * K


def _sc_logits_body(emb_hbm, src_hbm, dst_hbm, out_hbm,
                    sidx, didx, srows, drows, lg, sem_s, sem_d):
    wid = lax.axis_index("s") * NC + lax.axis_index("c")
    base0 = wid * EPW

    def chunk_body(k, carry):
        base = base0 + k * C
        pltpu.sync_copy(src_hbm.at[pl.ds(base, C)], sidx)
        pltpu.sync_copy(dst_hbm.at[pl.ds(base, C)], didx)
        g_s = pltpu.async_copy(emb_hbm.at[sidx], srows, sem_s)
        g_d = pltpu.async_copy(emb_hbm.at[didx], drows, sem_d)
        g_s.wait()
        g_d.wait()

        iota = lax.iota(jnp.int32, L)

        def group_body(g, carry2):
            res = jnp.zeros((L,), jnp.float32)
            for j in range(L):
                e = g * L + j
                acc = srows[e, pl.ds(0, L)] * drows[e, pl.ds(0, L)]
                for f in range(1, D // L):
                    acc = acc + (srows[e, pl.ds(f * L, L)]
                                 * drows[e, pl.ds(f * L, L)])
                res = jnp.where(iota == j, jnp.sum(acc), res)
            lg[pl.ds(g * L, L)] = res
            return carry2

        lax.fori_loop(0, C // L, group_body, 0)
        pltpu.sync_copy(lg, out_hbm.at[pl.ds(base, C)])
        return carry

    lax.fori_loop(0, K, chunk_body, 0)


_sc_logits = functools.partial(
    pl.kernel,
    out_type=jax.ShapeDtypeStruct((BP,), jnp.float32),
    mesh=plsc.VectorSubcoreMesh(
        core_axis_name="c", subcore_axis_name="s",
        num_cores=NC, num_subcores=NS),
    compiler_params=pltpu.CompilerParams(needs_layout_passes=False),
    scratch_types=[
        pltpu.VMEM((C,), jnp.int32),
        pltpu.VMEM((C,), jnp.int32),
        pltpu.VMEM((C, D), jnp.float32),
        pltpu.VMEM((C, D), jnp.float32),
        pltpu.VMEM((C,), jnp.float32),
        pltpu.SemaphoreType.DMA,
        pltpu.SemaphoreType.DMA,
    ],
)(_sc_logits_body)


def _bce_body(x_ref, y_ref, o_ref):
    x = x_ref[...]
    y = y_ref[...]
    rows, cols = x.shape
    lin = (lax.broadcasted_iota(jnp.int32, (rows, cols), 0) * cols
           + lax.broadcasted_iota(jnp.int32, (rows, cols), 1))
    elt = jnp.maximum(x, 0.0) - x * y + jnp.log1p(jnp.exp(-jnp.abs(x)))
    elt = jnp.where(lin < B, elt, 0.0)
    o_ref[...] = (jnp.sum(elt) / B).reshape(1, 1)


def kernel(node_emb, edge_index, edge_label):
    pad = BP - B
    src = jnp.pad(edge_index[:, 0], (0, pad))
    dst = jnp.pad(edge_index[:, 1], (0, pad))

    logits = _sc_logits(node_emb, src, dst)

    rows = BP // 128
    logits2d = logits.reshape(rows, 128)
    labels2d = jnp.pad(edge_label, (0, pad)).reshape(rows, 128)

    loss = pl.pallas_call(
        _bce_body,
        out_shape=jax.ShapeDtypeStruct((1, 1), jnp.float32),
    )(logits2d, labels2d)
    return loss[0, 0]
